# Initial kernel scaffold; baseline (speedup 1.0000x reference)
import jax
import jax.numpy as jnp
from jax.experimental import pallas as pl


def kernel(x, img):
    raise NotImplementedError("write your pallas kernel here")


# speedup vs baseline: 27.6307x; 27.6307x over previous
import functools

import jax
import jax.numpy as jnp
from jax import lax
from jax.experimental import pallas as pl
from jax.experimental.pallas import tpu as pltpu
from jax.experimental.pallas import tpu_sc as plsc

_C, _H, _W = 3, 512, 512
_N = 1048576
_NC, _NS = 2, 16
_NW = _NC * _NS
_PW = _N // _NW
_B = 2048
_NG = _B // 16
_NCHUNK = _PW // _B

_mesh = plsc.VectorSubcoreMesh(
    core_axis_name="c", subcore_axis_name="s",
    num_cores=_NC, num_subcores=_NS)


def _build_quad(img):
    x1 = jnp.concatenate([img[:, :, 1:], img[:, :, -1:]], axis=2)
    y1 = jnp.concatenate([img[:, 1:, :], img[:, -1:, :]], axis=1)
    xy1 = jnp.concatenate([y1[:, :, 1:], y1[:, :, -1:]], axis=2)
    q = jnp.stack([img, x1, y1, xy1], axis=0)
    q = jnp.pad(q, ((0, 0), (0, 1), (0, 0), (0, 0)))
    return q.transpose(2, 3, 0, 1).reshape(_H * _W, 16)


@functools.partial(
    pl.kernel,
    out_type=jax.ShapeDtypeStruct((_N * 3,), jnp.float32),
    mesh=_mesh,
    compiler_params=pltpu.CompilerParams(
        needs_layout_passes=False, use_tc_tiling_on_sc=False),
    scratch_types=[
        pltpu.VMEM((2 * _B,), jnp.float32),
        pltpu.VMEM((_B,), jnp.int32),
        pltpu.VMEM((4 * _B,), jnp.float32),
        pltpu.VMEM((_B, 16), jnp.float32),
        pltpu.VMEM((3 * _B,), jnp.float32),
        pltpu.SemaphoreType.DMA,
    ],
)
def _interp(xy_hbm, quad_hbm, out_hbm, xy_v, idx_v, w_v, q_v, out_v, sem):
    wid = lax.axis_index("s") * _NC + lax.axis_index("c")
    base_pt = wid * _PW
    iota = jnp.arange(16, dtype=jnp.int32)
    iota2 = iota * 2
    dpat = []
    cpat = []
    for m in range(3):
        q16 = iota + 16 * m
        dpat.append(q16 // 3)
        cpat.append(q16 % 3)

    def chunk_body(g, carry):
        cbase = base_pt + g * _B
        pltpu.sync_copy(xy_hbm.at[pl.ds(cbase * 2, 2 * _B)], xy_v)

        def p1(i, carry1):
            o = i * 32
            xv = plsc.load_gather(xy_v, [iota2 + o])
            yv = plsc.load_gather(xy_v, [iota2 + (o + 1)])
            ix = ((xv * 2.0 - 1.0) + 1.0) * (0.5 * (_W - 1))
            iy = ((yv * 2.0 - 1.0) + 1.0) * (0.5 * (_H - 1))
            ix = jnp.clip(ix, 0.0, float(_W - 1))
            iy = jnp.clip(iy, 0.0, float(_H - 1))
            x0 = ix.astype(jnp.int32)
            y0 = iy.astype(jnp.int32)
            wx = ix - x0.astype(jnp.float32)
            wy = iy - y0.astype(jnp.float32)
            s = i * 16
            idx_v[pl.ds(s, 16)] = y0 * _W + x0
            wxm = 1.0 - wx
            wym = 1.0 - wy
            w_v[pl.ds(s, 16)] = wxm * wym
            w_v[pl.ds(_B + s, 16)] = wx * wym
            w_v[pl.ds(2 * _B + s, 16)] = wxm * wy
            w_v[pl.ds(3 * _B + s, 16)] = wx * wy
            return carry1

        lax.fori_loop(0, _NG, p1, 0)

        pltpu.async_copy(quad_hbm.at[idx_v], q_v, sem).wait()

        def p2(i, carry2):
            s = i * 16
            for m in range(3):
                pt = dpat[m] + s
                acc = None
                for k in range(4):
                    wk = plsc.load_gather(w_v, [pt + k * _B])
                    vk = plsc.load_gather(q_v, [pt, cpat[m] + 4 * k])
                    acc = wk * vk if acc is None else acc + wk * vk
                out_v[pl.ds(i * 48 + m * 16, 16)] = acc
            return carry2

        lax.fori_loop(0, _NG, p2, 0)

        pltpu.sync_copy(out_v, out_hbm.at[pl.ds(cbase * 3, 3 * _B)])
        return carry

    lax.fori_loop(0, _NCHUNK, chunk_body, 0)


def kernel(x, img):
    xy = x.reshape(-1)
    quad = _build_quad(img)
    out = _interp(xy, quad)
    return out.reshape(_N, 3)


# speedup vs baseline: 31.5035x; 1.1402x over previous
import functools

import jax
import jax.numpy as jnp
from jax import lax
from jax.experimental import pallas as pl
from jax.experimental.pallas import tpu as pltpu
from jax.experimental.pallas import tpu_sc as plsc

_C, _H, _W = 3, 512, 512
_N = 1048576
_NC, _NS = 2, 16
_NW = _NC * _NS
_PW = _N // _NW
_B = 2048
_NG = _B // 16
_NCHUNK = _PW // _B
_RPW = _H // _NW

_mesh = plsc.VectorSubcoreMesh(
    core_axis_name="c", subcore_axis_name="s",
    num_cores=_NC, num_subcores=_NS)

_params = pltpu.CompilerParams(
    needs_layout_passes=False, use_tc_tiling_on_sc=False)


@functools.partial(
    pl.kernel,
    out_type=jax.ShapeDtypeStruct((_H * _W, 16), jnp.float32),
    mesh=_mesh,
    compiler_params=_params,
    scratch_types=[
        pltpu.VMEM((_C, _RPW + 1, _W), jnp.float32),
        pltpu.VMEM((_W, 16), jnp.float32),
        pltpu.SemaphoreType.DMA,
    ],
)
def _build(img_hbm, quad_hbm, rbuf, qrow, sem):
    wid = lax.axis_index("s") * _NC + lax.axis_index("c")
    y0 = wid * _RPW
    iota = jnp.arange(16, dtype=jnp.int32)
    c_pat = jnp.minimum(iota & 3, 2)
    dy_pat = (iota >> 2) >> 1
    dx_pat = (iota >> 2) & 1

    for c in range(_C):
        pltpu.sync_copy(img_hbm.at[c, pl.ds(y0, _RPW), :],
                        rbuf.at[c, pl.ds(0, _RPW), :])
    y17 = jnp.minimum(y0 + _RPW, _H - 1)
    for c in range(_C):
        pltpu.sync_copy(img_hbm.at[c, pl.ds(y17, 1), :],
                        rbuf.at[c, pl.ds(_RPW, 1), :])

    def row_body(yl, carry):
        y_pat = dy_pat + yl

        def x_body(xg, carry2):
            for u in range(8):
                xx = xg * 8 + u
                v = plsc.load_gather(rbuf, [c_pat, y_pat, dx_pat + xx])
                plsc.store_scatter(qrow, [jnp.full((16,), xx, jnp.int32),
                                          iota], v)
            return carry2

        lax.fori_loop(0, _W // 8, x_body, 0)
        v = plsc.load_gather(rbuf, [c_pat, y_pat,
                                    jnp.full((16,), _W - 1, jnp.int32)])
        plsc.store_scatter(qrow, [jnp.full((16,), _W - 1, jnp.int32), iota], v)
        pltpu.sync_copy(qrow, quad_hbm.at[pl.ds((y0 + yl) * _W, _W)])
        return carry

    lax.fori_loop(0, _RPW, row_body, 0)


@functools.partial(
    pl.kernel,
    out_type=jax.ShapeDtypeStruct((_N, 3), jnp.float32),
    mesh=_mesh,
    compiler_params=_params,
    scratch_types=[
        pltpu.VMEM((2 * _B,), jnp.float32),
        pltpu.VMEM((_B,), jnp.int32),
        pltpu.VMEM((4 * _B,), jnp.float32),
        pltpu.VMEM((_B, 16), jnp.float32),
        pltpu.VMEM((_B, 3), jnp.float32),
        pltpu.SemaphoreType.DMA,
    ],
)
def _interp(xy_hbm, quad_hbm, out_hbm, xy_v, idx_v, w_v, q_v, out_v, sem):
    wid = lax.axis_index("s") * _NC + lax.axis_index("c")
    base_pt = wid * _PW
    iota = jnp.arange(16, dtype=jnp.int32)
    iota2 = iota * 2
    dpat = []
    cpat = []
    for m in range(3):
        q16 = iota + 16 * m
        dpat.append(q16 // 3)
        cpat.append(q16 % 3)

    def chunk_body(g, carry):
        cbase = base_pt + g * _B
        pltpu.sync_copy(xy_hbm.at[pl.ds(cbase * 2, 2 * _B)], xy_v)

        def p1(i, carry1):
            o = i * 32
            xv = plsc.load_gather(xy_v, [iota2 + o])
            yv = plsc.load_gather(xy_v, [iota2 + (o + 1)])
            ix = ((xv * 2.0 - 1.0) + 1.0) * (0.5 * (_W - 1))
            iy = ((yv * 2.0 - 1.0) + 1.0) * (0.5 * (_H - 1))
            ix = jnp.clip(ix, 0.0, float(_W - 1))
            iy = jnp.clip(iy, 0.0, float(_H - 1))
            x0 = ix.astype(jnp.int32)
            y0 = iy.astype(jnp.int32)
            wx = ix - x0.astype(jnp.float32)
            wy = iy - y0.astype(jnp.float32)
            s = i * 16
            idx_v[pl.ds(s, 16)] = y0 * _W + x0
            wxm = 1.0 - wx
            wym = 1.0 - wy
            w_v[pl.ds(s, 16)] = wxm * wym
            w_v[pl.ds(_B + s, 16)] = wx * wym
            w_v[pl.ds(2 * _B + s, 16)] = wxm * wy
            w_v[pl.ds(3 * _B + s, 16)] = wx * wy
            return carry1

        lax.fori_loop(0, _NG, p1, 0)

        pltpu.async_copy(quad_hbm.at[idx_v], q_v, sem).wait()

        def p2(i, carry2):
            s = i * 16
            for m in range(3):
                pt = dpat[m] + s
                acc = None
                for k in range(4):
                    wk = plsc.load_gather(w_v, [pt + k * _B])
                    vk = plsc.load_gather(q_v, [pt, cpat[m] + 4 * k])
                    acc = wk * vk if acc is None else acc + wk * vk
                plsc.store_scatter(out_v, [pt, cpat[m]], acc)
            return carry2

        lax.fori_loop(0, _NG, p2, 0)

        pltpu.sync_copy(out_v, out_hbm.at[pl.ds(cbase, _B)])
        return carry

    lax.fori_loop(0, _NCHUNK, chunk_body, 0)


def kernel(x, img):
    quad = _build(img)
    return _interp(x.reshape(-1), quad)


# speedup vs baseline: 90.4663x; 2.8716x over previous
import functools

import jax
import jax.numpy as jnp
from jax import lax
from jax.experimental import pallas as pl
from jax.experimental.pallas import tpu as pltpu
from jax.experimental.pallas import tpu_sc as plsc

_C, _H, _W = 3, 512, 512
_N = 1048576
_NC, _NS = 2, 16
_NW = _NC * _NS
_PW = _N // _NW
_B = 2048
_NG = _B // 16
_NCHUNK = _PW // _B
_RPW = _H // _NW

_mesh = plsc.VectorSubcoreMesh(
    core_axis_name="c", subcore_axis_name="s",
    num_cores=_NC, num_subcores=_NS)

_params = pltpu.CompilerParams(
    needs_layout_passes=False, use_tc_tiling_on_sc=False)


@functools.partial(
    pl.kernel,
    out_type=jax.ShapeDtypeStruct((_H * _W, 16), jnp.float32),
    mesh=_mesh,
    compiler_params=_params,
    scratch_types=[
        pltpu.VMEM((_C, _RPW + 1, _W), jnp.float32),
        pltpu.VMEM((_W, 16), jnp.float32),
        pltpu.SemaphoreType.DMA,
    ],
)
def _build(img_hbm, quad_hbm, rbuf, qrow, sem):
    wid = lax.axis_index("s") * _NC + lax.axis_index("c")
    y0 = wid * _RPW
    iota = jnp.arange(16, dtype=jnp.int32)
    c_pat = jnp.minimum(iota & 3, 2)
    dy_pat = (iota >> 2) >> 1
    dx_pat = (iota >> 2) & 1

    for c in range(_C):
        pltpu.sync_copy(img_hbm.at[c, pl.ds(y0, _RPW), :],
                        rbuf.at[c, pl.ds(0, _RPW), :])
    y17 = jnp.minimum(y0 + _RPW, _H - 1)
    for c in range(_C):
        pltpu.sync_copy(img_hbm.at[c, pl.ds(y17, 1), :],
                        rbuf.at[c, pl.ds(_RPW, 1), :])

    def row_body(yl, carry):
        y_pat = dy_pat + yl

        def x_body(xg, carry2):
            for u in range(8):
                xx = xg * 8 + u
                v = plsc.load_gather(rbuf, [c_pat, y_pat, dx_pat + xx])
                plsc.store_scatter(qrow, [jnp.full((16,), xx, jnp.int32),
                                          iota], v)
            return carry2

        lax.fori_loop(0, _W // 8, x_body, 0)
        v = plsc.load_gather(rbuf, [c_pat, y_pat,
                                    jnp.full((16,), _W - 1, jnp.int32)])
        plsc.store_scatter(qrow, [jnp.full((16,), _W - 1, jnp.int32), iota], v)
        pltpu.sync_copy(qrow, quad_hbm.at[pl.ds((y0 + yl) * _W, _W)])
        return carry

    lax.fori_loop(0, _RPW, row_body, 0)


@functools.partial(
    pl.kernel,
    out_type=jax.ShapeDtypeStruct((_N, 3), jnp.float32),
    mesh=_mesh,
    compiler_params=_params,
    scratch_types=[
        pltpu.VMEM((_B,), jnp.float32),
        pltpu.VMEM((_B,), jnp.float32),
        pltpu.VMEM((2 * _B,), jnp.int32),
        pltpu.VMEM((8 * _B,), jnp.float32),
        pltpu.VMEM((2 * _B, 16), jnp.float32),
        pltpu.VMEM((_B, 3), jnp.float32),
        pltpu.SemaphoreType.DMA,
    ],
)
def _interp(xs_hbm, ys_hbm, quad_hbm, out_hbm,
            xs_v, ys_v, idx_v, w_v, q_v, out_v, sem):
    wid = lax.axis_index("s") * _NC + lax.axis_index("c")
    base_pt = wid * _PW
    iota = jnp.arange(16, dtype=jnp.int32)
    dpat = []
    cpat = []
    for m in range(3):
        q16 = iota + 16 * m
        dpat.append(q16 // 3)
        cpat.append(q16 % 3)

    def p1(g, par):
        cbase = base_pt + g * _B
        pltpu.sync_copy(xs_hbm.at[pl.ds(cbase, _B)], xs_v)
        pltpu.sync_copy(ys_hbm.at[pl.ds(cbase, _B)], ys_v)
        ib = par * _B
        wb = par * 4 * _B

        def body(i, carry1):
            for u in range(2):
                ii = i * 2 + u
                s0 = ii * 16
                xv = xs_v[pl.ds(s0, 16)]
                yv = ys_v[pl.ds(s0, 16)]
                ix = ((xv * 2.0 - 1.0) + 1.0) * (0.5 * (_W - 1))
                iy = ((yv * 2.0 - 1.0) + 1.0) * (0.5 * (_H - 1))
                ix = jnp.clip(ix, 0.0, float(_W - 1))
                iy = jnp.clip(iy, 0.0, float(_H - 1))
                x0 = ix.astype(jnp.int32)
                y0 = iy.astype(jnp.int32)
                wx = ix - x0.astype(jnp.float32)
                wy = iy - y0.astype(jnp.float32)
                s = s0
                idx_v[pl.ds(ib + s, 16)] = y0 * _W + x0
                wxm = 1.0 - wx
                wym = 1.0 - wy
                w_v[pl.ds(wb + s, 16)] = wxm * wym
                w_v[pl.ds(wb + _B + s, 16)] = wx * wym
                w_v[pl.ds(wb + 2 * _B + s, 16)] = wxm * wy
                w_v[pl.ds(wb + 3 * _B + s, 16)] = wx * wy
            return carry1

        lax.fori_loop(0, _NG // 2, body, 0)
        pltpu.async_copy(quad_hbm.at[idx_v.at[pl.ds(par * _B, _B)]],
                         q_v.at[pl.ds(par * _B, _B)], sem)

    def drain(par):
        pltpu.make_async_copy(quad_hbm.at[idx_v.at[pl.ds(par * _B, _B)]],
                              q_v.at[pl.ds(par * _B, _B)], sem).wait()

    def p2(g, par):
        cbase = base_pt + g * _B
        qb = par * _B
        wb = par * 4 * _B

        def body(i, carry2):
            for u in range(2):
                s = (i * 2 + u) * 16
                for m in range(3):
                    pt = dpat[m] + s
                    acc = None
                    for k in range(4):
                        wk = plsc.load_gather(w_v, [pt + (wb + k * _B)])
                        vk = plsc.load_gather(q_v, [pt + qb, cpat[m] + 4 * k])
                        acc = wk * vk if acc is None else acc + wk * vk
                    plsc.store_scatter(out_v, [pt, cpat[m]], acc)
            return carry2

        lax.fori_loop(0, _NG // 2, body, 0)
        pltpu.sync_copy(out_v, out_hbm.at[pl.ds(cbase, _B)])

    p1(0, 0)

    def pair_body(h, carry):
        g0 = h * 2
        p1(g0 + 1, 1)
        drain(0)
        p2(g0, 0)
        p1(g0 + 2, 0)
        drain(1)
        p2(g0 + 1, 1)
        return carry

    lax.fori_loop(0, _NCHUNK // 2 - 1, pair_body, 0)
    gl = _NCHUNK - 2
    p1(gl + 1, 1)
    drain(0)
    p2(gl, 0)
    drain(1)
    p2(gl + 1, 1)


def kernel(x, img):
    quad = _build(img)
    return _interp(x[:, 0], x[:, 1], quad)
